# baseline (speedup 1.0000x reference)
import functools

import jax
import jax.numpy as jnp
from jax import lax
from jax.experimental import pallas as pl
from jax.experimental.pallas import tpu as pltpu
from jax.experimental.pallas import tpu_sc as plsc

EMB = 64
CHUNK = 128
NBUF = 2
NC = 2
NS = 16
NW = NC * NS


@functools.lru_cache(maxsize=None)
def _make_kernel(n_flat: int):
    per_w = n_flat // NW
    nchunks = per_w // CHUNK
    assert per_w * NW == n_flat and nchunks * CHUNK == per_w

    mesh = plsc.VectorSubcoreMesh(core_axis_name="c", subcore_axis_name="s")

    @functools.partial(
        pl.kernel,
        mesh=mesh,
        out_type=jax.ShapeDtypeStruct((n_flat, EMB), jnp.float32),
        compiler_params=pltpu.CompilerParams(use_tc_tiling_on_sc=False),
        scratch_types=[
            pltpu.VMEM((per_w,), jnp.int32),
            pltpu.VMEM((CHUNK, EMB), jnp.float32),
            pltpu.VMEM((CHUNK, EMB), jnp.float32),
            pltpu.VMEM((EMB,), jnp.float32),
            pltpu.SemaphoreType.DMA,
            pltpu.SemaphoreType.DMA,
        ],
    )
    def body(x_hbm, wemb_hbm, wpos_hbm, out_hbm, idx_v, buf0, buf1, wpos_v,
             sem0, sem1):
        wid = lax.axis_index("s") * NC + lax.axis_index("c")
        pltpu.sync_copy(x_hbm.at[pl.ds(wid * per_w, per_w)], idx_v)
        pltpu.sync_copy(wpos_hbm, wpos_v)
        wp = [wpos_v[pl.ds(16 * q, 16)] for q in range(4)]
        bufs = (buf0, buf1)
        sems = (sem0, sem1)
        out_base = wid * per_w

        def issue(j, b):
            idx = idx_v.at[pl.ds(pl.multiple_of(j * CHUNK, 8), CHUNK)]
            pltpu.async_copy(wemb_hbm.at[idx], bufs[b], sems[b])

        def wait(b):
            idx = idx_v.at[pl.ds(0, CHUNK)]
            pltpu.make_async_copy(wemb_hbm.at[idx], bufs[b], sems[b]).wait()

        def process(j, b):
            buf = bufs[b]

            def addrow(r, carry):
                for q in range(4):
                    sl = pl.ds(16 * q, 16)
                    buf[r, sl] = buf[r, sl] + wp[q]
                return carry

            lax.fori_loop(0, CHUNK, addrow, 0, unroll=2)
            pltpu.sync_copy(buf, out_hbm.at[pl.ds(out_base + j * CHUNK, CHUNK)])

        for b in range(NBUF):
            issue(b, b)

        @pl.loop(0, (nchunks - NBUF) // NBUF)
        def main(i):
            j0 = i * NBUF
            for b in range(NBUF):
                j = j0 + b
                wait(b)
                process(j, b)
                issue(j + NBUF, b)

        for b in range(NBUF):
            j = nchunks - NBUF + b
            wait(b)
            process(j, b)

    return body


def kernel(x, W_emb, W_pos):
    b, l = x.shape
    n = b * l
    xf = x.reshape(n).astype(jnp.int32)
    out = _make_kernel(n)(xf, W_emb, W_pos)
    return out.reshape(b, l, EMB)


# speedup vs baseline: 1.0108x; 1.0108x over previous
import functools

import jax
import jax.numpy as jnp
from jax import lax
from jax.experimental import pallas as pl
from jax.experimental.pallas import tpu as pltpu
from jax.experimental.pallas import tpu_sc as plsc

EMB = 64
ROWBLK = 8
NBUF = 2
NC = 2
NS = 16
NW = NC * NS


@functools.lru_cache(maxsize=None)
def _make_kernel(batch: int, seq: int):
    rows_w = batch // NW
    nsteps = rows_w // ROWBLK
    assert rows_w * NW == batch and nsteps * ROWBLK == rows_w
    nq = EMB // 16

    mesh = plsc.VectorSubcoreMesh(core_axis_name="c", subcore_axis_name="s")

    @functools.partial(
        pl.kernel,
        mesh=mesh,
        out_type=jax.ShapeDtypeStruct((batch, seq, EMB), jnp.float32),
        compiler_params=pltpu.CompilerParams(use_tc_tiling_on_sc=False),
        scratch_types=[
            pltpu.VMEM((rows_w, seq), jnp.int32),
            pltpu.VMEM((ROWBLK, seq, EMB), jnp.float32),
            pltpu.VMEM((ROWBLK, seq, EMB), jnp.float32),
            pltpu.VMEM((EMB,), jnp.float32),
            pltpu.SemaphoreType.DMA,
            pltpu.SemaphoreType.DMA,
        ],
    )
    def body(x_hbm, wemb_hbm, wpos_hbm, out_hbm, idx_v, buf0, buf1, wpos_v,
             sem0, sem1):
        wid = lax.axis_index("s") * NC + lax.axis_index("c")
        row0 = wid * rows_w
        pltpu.sync_copy(x_hbm.at[pl.ds(row0, rows_w)], idx_v)
        pltpu.sync_copy(wpos_hbm, wpos_v)
        wp = [wpos_v[pl.ds(16 * q, 16)] for q in range(nq)]
        bufs = (buf0, buf1)
        sems = (sem0, sem1)

        def issue(i, b):
            for a in range(ROWBLK):
                pltpu.async_copy(
                    wemb_hbm.at[idx_v.at[i * ROWBLK + a]],
                    bufs[b].at[a], sems[b])

        def wait(b):
            for a in range(ROWBLK):
                pltpu.make_async_copy(
                    wemb_hbm.at[idx_v.at[0]], bufs[b].at[a], sems[b]).wait()

        def process(i, b):
            buf = bufs[b]

            def addrow(r, carry):
                for a in range(ROWBLK):
                    for q in range(nq):
                        sl = pl.ds(16 * q, 16)
                        buf[a, r, sl] = buf[a, r, sl] + wp[q]
                return carry

            lax.fori_loop(0, seq, addrow, 0)
            pltpu.sync_copy(
                buf, out_hbm.at[pl.ds(row0 + i * ROWBLK, ROWBLK)])

        for b in range(NBUF):
            issue(b, b)

        @pl.loop(0, (nsteps - NBUF) // NBUF)
        def main(k):
            i0 = k * NBUF
            for b in range(NBUF):
                i = i0 + b
                wait(b)
                process(i, b)
                issue(i + NBUF, b)

        for b in range(NBUF):
            i = nsteps - NBUF + b
            wait(b)
            process(i, b)

    return body


def kernel(x, W_emb, W_pos):
    b, l = x.shape
    return _make_kernel(b, l)(x.astype(jnp.int32), W_emb, W_pos)
